# Initial kernel scaffold; baseline (speedup 1.0000x reference)
import jax
import jax.numpy as jnp
from jax.experimental import pallas as pl


def kernel(left_x, right_x, left_edge_index, left_edge_type, right_edge_index, right_edge_type, varindex1, varindex2, W_left, root_left, bias_left, W_right, root_right, bias_right, ln_gamma, ln_beta):
    raise NotImplementedError("write your pallas kernel here")


# speedup vs baseline: 10.9978x; 10.9978x over previous
import functools

import jax
import jax.numpy as jnp
from jax import lax
from jax.experimental import pallas as pl
from jax.experimental.pallas import tpu as pltpu
from jax.experimental.pallas import tpu_sc as plsc

NC, NS = 2, 16
NW = NC * NS
CHUNK = 128


def _tc_transform(x, W):
    N, C = x.shape
    R = W.shape[0]
    BN = 400

    def body(x_ref, w_ref, y_ref):
        xb = x_ref[...]
        for r in range(R):
            y_ref[r] = jnp.dot(xb, w_ref[r], preferred_element_type=jnp.float32)

    return pl.pallas_call(
        body,
        grid=(N // BN,),
        in_specs=[
            pl.BlockSpec((BN, C), lambda i: (i, 0)),
            pl.BlockSpec((R, C, C), lambda i: (0, 0, 0)),
        ],
        out_specs=pl.BlockSpec((R, BN, C), lambda i: (0, i, 0)),
        out_shape=jax.ShapeDtypeStruct((R, N, C), jnp.float32),
    )(x, W)


def _tc_combine(x, parts, root, bias2d, W, g2d, b2d, want_y):
    N, C = x.shape
    R = W.shape[0]
    BN = 400

    def body(x_ref, p_ref, root_ref, bias_ref, w_ref, g_ref, b_ref, xo_ref,
             *maybe_y):
        h = jnp.dot(x_ref[...], root_ref[...], preferred_element_type=jnp.float32)
        h = h + bias_ref[...] + p_ref[0] + p_ref[1]
        m = jnp.mean(h, axis=-1, keepdims=True)
        v = jnp.mean((h - m) ** 2, axis=-1, keepdims=True)
        hn = (h - m) / jnp.sqrt(v + 1e-5) * g_ref[...] + b_ref[...]
        xo = jnp.maximum(hn, 0.0)
        xo_ref[...] = xo
        if want_y:
            y_ref = maybe_y[0]
            for r in range(R):
                y_ref[r] = jnp.dot(xo, w_ref[r], preferred_element_type=jnp.float32)

    out_shape = [jax.ShapeDtypeStruct((N, C), jnp.float32)]
    out_specs = [pl.BlockSpec((BN, C), lambda i: (i, 0))]
    if want_y:
        out_shape.append(jax.ShapeDtypeStruct((R, N, C), jnp.float32))
        out_specs.append(pl.BlockSpec((R, BN, C), lambda i: (0, i, 0)))

    res = pl.pallas_call(
        body,
        grid=(N // BN,),
        in_specs=[
            pl.BlockSpec((BN, C), lambda i: (i, 0)),
            pl.BlockSpec((2, BN, C), lambda i: (0, i, 0)),
            pl.BlockSpec((C, C), lambda i: (0, 0)),
            pl.BlockSpec((1, C), lambda i: (0, 0)),
            pl.BlockSpec((R, C, C), lambda i: (0, 0, 0)),
            pl.BlockSpec((1, C), lambda i: (0, 0)),
            pl.BlockSpec((1, C), lambda i: (0, 0)),
        ],
        out_specs=out_specs,
        out_shape=out_shape,
    )(x, parts, root, bias2d, W, g2d, b2d)
    return res if want_y else (res[0], None)


def _tc_dot(v1, v2):
    Vn, C = v1.shape

    def body(a_ref, b_ref, o_ref):
        o_ref[...] = lax.dot_general(
            a_ref[...], b_ref[...], (((1,), (1,)), ((), ())),
            preferred_element_type=jnp.float32)

    return pl.pallas_call(
        body, out_shape=jax.ShapeDtypeStruct((Vn, Vn), jnp.float32))(v1, v2)


def _sc_mesh():
    return plsc.VectorSubcoreMesh(core_axis_name="c", subcore_axis_name="s")


def _vec_iota():
    return lax.iota(jnp.int32, 16)


def _sc_prep(src, dst, etype, n_nodes, n_rel):
    E = src.shape[0]
    PAD = 53248
    assert n_rel * n_nodes <= PAD
    EPT = E // NS
    n_full_cnt, tail_cnt = EPT // CHUNK, EPT % CHUNK
    EPW = E // NW
    n_full, tail = EPW // CHUNK, EPW % CHUNK
    assert tail_cnt % 8 == 0 and tail % 8 == 0
    RPT = PAD // NS

    ZCH = RPT // 2

    def body(src_h, dst_h, typ_h, gidx_h, invc_h, cnt_sh, ones_b, dst_b, typ_b,
             src_b, cidx_b, cidx_tc, cidx_te, gidx_b, vals_b, out_b, zrow_b,
             sem):
        c = lax.axis_index("c")
        s = lax.axis_index("s")
        w = s * NC + c

        ones = jnp.ones((16,), jnp.float32)
        zv = jnp.zeros((16,), jnp.float32)

        @pl.loop(0, CHUNK // 16)
        def _fill1(i):
            ones_b[pl.ds(i * 16, 16)] = ones

        @pl.loop(0, ZCH // 16)
        def _fill0(i):
            zrow_b[pl.ds(i * 16, 16)] = zv

        @pl.loop(0, 2)
        def _zero(i):
            pltpu.sync_copy(zrow_b, cnt_sh.at[pl.ds(s * RPT + i * ZCH, ZCH)])

        plsc.subcore_barrier()

        def count_chunk(base, nb, ones_ref, idx_ref):
            pltpu.sync_copy(dst_h.at[pl.ds(base, nb)], dst_b.at[pl.ds(0, nb)])
            pltpu.sync_copy(typ_h.at[pl.ds(base, nb)], typ_b.at[pl.ds(0, nb)])
            for t in range(nb // 16):
                sl = pl.ds(t * 16, 16)
                idx_ref[sl] = typ_b[sl] * n_nodes + dst_b[sl]
            pltpu.sync_copy(ones_ref, cnt_sh.at[idx_ref], add=True)

        @pl.loop(0, n_full_cnt)
        def _count(i):
            count_chunk(s * EPT + i * CHUNK, CHUNK, ones_b, cidx_b)

        if tail_cnt:
            count_chunk(s * EPT + n_full_cnt * CHUNK, tail_cnt,
                        ones_b.at[pl.ds(0, tail_cnt)], cidx_tc)

        plsc.subcore_barrier()

        def emit_chunk(base, nb, idx_ref):
            pltpu.sync_copy(src_h.at[pl.ds(base, nb)], src_b.at[pl.ds(0, nb)])
            pltpu.sync_copy(dst_h.at[pl.ds(base, nb)], dst_b.at[pl.ds(0, nb)])
            pltpu.sync_copy(typ_h.at[pl.ds(base, nb)], typ_b.at[pl.ds(0, nb)])
            for t in range(nb // 16):
                sl = pl.ds(t * 16, 16)
                gidx_b[sl] = typ_b[sl] * n_nodes + src_b[sl]
                idx_ref[sl] = typ_b[sl] * n_nodes + dst_b[sl]
            pltpu.sync_copy(gidx_b.at[pl.ds(0, nb)], gidx_h.at[pl.ds(base, nb)])
            pltpu.async_copy(cnt_sh.at[idx_ref], vals_b.at[pl.ds(0, nb)],
                             sem).wait()
            for t in range(nb // 16):
                sl = pl.ds(t * 16, 16)
                out_b[sl] = 1.0 / jnp.maximum(vals_b[sl], 1.0)
            pltpu.sync_copy(out_b.at[pl.ds(0, nb)], invc_h.at[pl.ds(base, nb)])

        @pl.loop(0, n_full)
        def _emit(i):
            emit_chunk(w * EPW + i * CHUNK, CHUNK, cidx_b)

        if tail:
            emit_chunk(w * EPW + n_full * CHUNK, tail, cidx_te)

    kern = pl.kernel(
        body,
        out_type=(jax.ShapeDtypeStruct((E,), jnp.int32),
                  jax.ShapeDtypeStruct((E,), jnp.float32)),
        mesh=_sc_mesh(),
        scratch_types=(
            pltpu.VMEM_SHARED((PAD,), jnp.float32),
            pltpu.VMEM((CHUNK,), jnp.float32),
            pltpu.VMEM((CHUNK,), jnp.int32),
            pltpu.VMEM((CHUNK,), jnp.int32),
            pltpu.VMEM((CHUNK,), jnp.int32),
            pltpu.VMEM((CHUNK,), jnp.int32),
            pltpu.VMEM((max(tail_cnt, 8),), jnp.int32),
            pltpu.VMEM((max(tail, 8),), jnp.int32),
            pltpu.VMEM((CHUNK,), jnp.int32),
            pltpu.VMEM((CHUNK,), jnp.float32),
            pltpu.VMEM((CHUNK,), jnp.float32),
            pltpu.VMEM(((PAD // NS) // 2,), jnp.float32),
            pltpu.SemaphoreType.DMA,
        ),
    )
    return kern(src, dst, etype)


def _sc_aggregate(Y2, gidx, dst, invc, n_nodes):
    RN, C = Y2.shape
    E = gidx.shape[0]
    EPW = E // NW
    n_full, tail = EPW // CHUNK, EPW % CHUNK
    assert tail % 8 == 0
    NPT8 = 632
    LAST = n_nodes - (NS - 1) * NPT8
    T_MAIN = NPT8 - 4 * CHUNK
    T_LAST = LAST - 4 * CHUNK
    assert T_MAIN > 0 and T_LAST > 0 and T_MAIN % 8 == 0 and T_LAST % 8 == 0

    def body(y_h, gidx_h, dst_h, invc_h, out_h, agg_sh, rows_b, rows_t, gidx_b,
             gidx_t, dst_b, dst_t, invc_b, zb, sem):
        c = lax.axis_index("c")
        s = lax.axis_index("s")
        w = s * NC + c
        zv = jnp.zeros((16,), jnp.float32)

        @pl.loop(0, CHUNK)
        def _fillz(i):
            for t in range(C // 16):
                zb[i, pl.ds(t * 16, 16)] = zv

        rbase = s * NPT8

        @pl.loop(0, 4)
        def _zero(i):
            pltpu.sync_copy(zb, agg_sh.at[pl.ds(rbase + i * CHUNK, CHUNK)])

        @pl.when(s < NS - 1)
        def _zt_main():
            pltpu.sync_copy(zb.at[pl.ds(0, T_MAIN)],
                            agg_sh.at[pl.ds(rbase + 4 * CHUNK, T_MAIN)])

        @pl.when(s == NS - 1)
        def _zt_last():
            pltpu.sync_copy(zb.at[pl.ds(0, T_LAST)],
                            agg_sh.at[pl.ds(rbase + 4 * CHUNK, T_LAST)])

        plsc.subcore_barrier()

        def do_chunk(base, nb, rows_ref, gidx_ref, didx_ref):
            pltpu.sync_copy(gidx_h.at[pl.ds(base, nb)], gidx_ref)
            pltpu.sync_copy(dst_h.at[pl.ds(base, nb)], didx_ref)
            pltpu.sync_copy(invc_h.at[pl.ds(base, nb)], invc_b.at[pl.ds(0, nb)])
            pltpu.async_copy(y_h.at[gidx_ref], rows_ref, sem).wait()

            for t in range(nb // 16):
                v = invc_b[pl.ds(t * 16, 16)]
                for jj in range(16):
                    sv = v[jj]
                    j = t * 16 + jj
                    for u in range(C // 16):
                        sl = pl.ds(u * 16, 16)
                        rows_ref[j, sl] = rows_ref[j, sl] * sv

            pltpu.sync_copy(rows_ref, agg_sh.at[didx_ref], add=True)

        @pl.loop(0, n_full)
        def _main(i):
            do_chunk(w * EPW + i * CHUNK, CHUNK, rows_b, gidx_b, dst_b)

        if tail:
            do_chunk(w * EPW + n_full * CHUNK, tail, rows_t, gidx_t, dst_t)

        plsc.subcore_barrier()

        def dump(off, nrows):
            pltpu.sync_copy(agg_sh.at[pl.ds(rbase + off, nrows)],
                            zb.at[pl.ds(0, nrows)])
            pltpu.sync_copy(zb.at[pl.ds(0, nrows)],
                            out_h.at[pl.ds(c * n_nodes + rbase + off, nrows)])

        @pl.loop(0, 4)
        def _dump(i):
            dump(i * CHUNK, CHUNK)

        @pl.when(s < NS - 1)
        def _dt_main():
            dump(4 * CHUNK, T_MAIN)

        @pl.when(s == NS - 1)
        def _dt_last():
            dump(4 * CHUNK, T_LAST)

    kern = pl.kernel(
        body,
        out_type=jax.ShapeDtypeStruct((2 * n_nodes, C), jnp.float32),
        mesh=_sc_mesh(),
        scratch_types=(
            pltpu.VMEM_SHARED((n_nodes, C), jnp.float32),
            pltpu.VMEM((CHUNK, C), jnp.float32),
            pltpu.VMEM((max(tail, 8), C), jnp.float32),
            pltpu.VMEM((CHUNK,), jnp.int32),
            pltpu.VMEM((max(tail, 8),), jnp.int32),
            pltpu.VMEM((CHUNK,), jnp.int32),
            pltpu.VMEM((max(tail, 8),), jnp.int32),
            pltpu.VMEM((CHUNK,), jnp.float32),
            pltpu.VMEM((CHUNK, C), jnp.float32),
            pltpu.SemaphoreType.DMA,
        ),
    )
    return kern(Y2, gidx, dst, invc)


def _sc_gather_rows(lx, rx, vi1, vi2):
    N, C = lx.shape
    Vn = vi1.shape[0]
    VPW = Vn // NW

    def body(lx_h, rx_h, vi1_h, vi2_h, o1_h, o2_h, ib, rb, sem):
        c = lax.axis_index("c")
        s = lax.axis_index("s")
        w = s * NC + c
        base = w * VPW
        pltpu.sync_copy(vi1_h.at[pl.ds(base, VPW)], ib)
        pltpu.async_copy(lx_h.at[ib], rb, sem).wait()
        pltpu.sync_copy(rb, o1_h.at[pl.ds(base, VPW)])
        pltpu.sync_copy(vi2_h.at[pl.ds(base, VPW)], ib)
        pltpu.async_copy(rx_h.at[ib], rb, sem).wait()
        pltpu.sync_copy(rb, o2_h.at[pl.ds(base, VPW)])

    kern = pl.kernel(
        body,
        out_type=(jax.ShapeDtypeStruct((Vn, C), jnp.float32),
                  jax.ShapeDtypeStruct((Vn, C), jnp.float32)),
        mesh=_sc_mesh(),
        scratch_types=(
            pltpu.VMEM((VPW,), jnp.int32),
            pltpu.VMEM((VPW, C), jnp.float32),
            pltpu.SemaphoreType.DMA,
        ),
    )
    return kern(lx, rx, vi1, vi2)


def _run_side(x, edge_index, edge_type, W, root, bias, g, b, rounds):
    N, C = x.shape
    R = W.shape[0]
    src = edge_index[0]
    dst = edge_index[1]
    gidx, invc = _sc_prep(src, dst, edge_type, N, R)
    bias2d = bias.reshape(1, C)
    g2d = g.reshape(1, C)
    b2d = b.reshape(1, C)
    Y = _tc_transform(x, W)
    for t in range(rounds):
        parts = _sc_aggregate(Y.reshape(R * N, C), gidx, dst, invc, N)
        x, Y = _tc_combine(x, parts.reshape(2, N, C), root, bias2d, W, g2d,
                           b2d, want_y=(t < rounds - 1))
    return x


def kernel(left_x, right_x, left_edge_index, left_edge_type, right_edge_index,
           right_edge_type, varindex1, varindex2, W_left, root_left, bias_left,
           W_right, root_right, bias_right, ln_gamma, ln_beta):
    rounds = 5
    lx = _run_side(left_x, left_edge_index, left_edge_type, W_left, root_left,
                   bias_left, ln_gamma, ln_beta, rounds)
    rx = _run_side(right_x, right_edge_index, right_edge_type, W_right,
                   root_right, bias_right, ln_gamma, ln_beta, rounds)
    v1, v2 = _sc_gather_rows(lx, rx, varindex1, varindex2)
    dots = _tc_dot(v1, v2)
    return (dots, lx)


# speedup vs baseline: 16.9385x; 1.5402x over previous
import functools

import jax
import jax.numpy as jnp
from jax import lax
from jax.experimental import pallas as pl
from jax.experimental.pallas import tpu as pltpu
from jax.experimental.pallas import tpu_sc as plsc

NC, NS = 2, 16
NW = NC * NS
CHUNK = 128


def _tc_transform(x, W):
    N, C = x.shape
    R = W.shape[0]
    BN = 400

    def body(x_ref, w_ref, y_ref):
        xb = x_ref[...]
        for r in range(R):
            y_ref[r] = jnp.dot(xb, w_ref[r], preferred_element_type=jnp.float32)

    return pl.pallas_call(
        body,
        grid=(N // BN,),
        in_specs=[
            pl.BlockSpec((BN, C), lambda i: (i, 0)),
            pl.BlockSpec((R, C, C), lambda i: (0, 0, 0)),
        ],
        out_specs=pl.BlockSpec((R, BN, C), lambda i: (0, i, 0)),
        out_shape=jax.ShapeDtypeStruct((R, N, C), jnp.float32),
    )(x, W)


def _tc_combine(x, parts, root, bias2d, W, g2d, b2d, want_y):
    N, C = x.shape
    R = W.shape[0]
    BN = 400

    def body(x_ref, p_ref, root_ref, bias_ref, w_ref, g_ref, b_ref, xo_ref,
             *maybe_y):
        h = jnp.dot(x_ref[...], root_ref[...], preferred_element_type=jnp.float32)
        h = h + bias_ref[...] + p_ref[0] + p_ref[1]
        m = jnp.mean(h, axis=-1, keepdims=True)
        v = jnp.mean((h - m) ** 2, axis=-1, keepdims=True)
        hn = (h - m) / jnp.sqrt(v + 1e-5) * g_ref[...] + b_ref[...]
        xo = jnp.maximum(hn, 0.0)
        xo_ref[...] = xo
        if want_y:
            y_ref = maybe_y[0]
            for r in range(R):
                y_ref[r] = jnp.dot(xo, w_ref[r], preferred_element_type=jnp.float32)

    out_shape = [jax.ShapeDtypeStruct((N, C), jnp.float32)]
    out_specs = [pl.BlockSpec((BN, C), lambda i: (i, 0))]
    if want_y:
        out_shape.append(jax.ShapeDtypeStruct((R, N, C), jnp.float32))
        out_specs.append(pl.BlockSpec((R, BN, C), lambda i: (0, i, 0)))

    res = pl.pallas_call(
        body,
        grid=(N // BN,),
        in_specs=[
            pl.BlockSpec((BN, C), lambda i: (i, 0)),
            pl.BlockSpec((2, BN, C), lambda i: (0, i, 0)),
            pl.BlockSpec((C, C), lambda i: (0, 0)),
            pl.BlockSpec((1, C), lambda i: (0, 0)),
            pl.BlockSpec((R, C, C), lambda i: (0, 0, 0)),
            pl.BlockSpec((1, C), lambda i: (0, 0)),
            pl.BlockSpec((1, C), lambda i: (0, 0)),
        ],
        out_specs=out_specs,
        out_shape=out_shape,
    )(x, parts, root, bias2d, W, g2d, b2d)
    return res if want_y else (res[0], None)


def _tc_dot(v1, v2):
    Vn, C = v1.shape

    def body(a_ref, b_ref, o_ref):
        o_ref[...] = lax.dot_general(
            a_ref[...], b_ref[...], (((1,), (1,)), ((), ())),
            preferred_element_type=jnp.float32)

    return pl.pallas_call(
        body, out_shape=jax.ShapeDtypeStruct((Vn, Vn), jnp.float32))(v1, v2)


def _sc_mesh():
    return plsc.VectorSubcoreMesh(core_axis_name="c", subcore_axis_name="s")


def _vec_iota():
    return lax.iota(jnp.int32, 16)


def _sc_prep(src, dst, etype, n_nodes, n_rel):
    E = src.shape[0]
    PAD = 53248
    assert n_rel * n_nodes <= PAD
    EPT = E // NS
    n_full_cnt, tail_cnt = EPT // CHUNK, EPT % CHUNK
    EPW = E // NW
    n_full, tail = EPW // CHUNK, EPW % CHUNK
    assert tail_cnt % 8 == 0 and tail % 8 == 0
    RPT = PAD // NS

    ZCH = RPT // 2

    def body(src_h, dst_h, typ_h, gidx_h, invc_h, cnt_sh, ones_b, dst_b, typ_b,
             src_b, cidx_b, cidx_tc, cidx_te, gidx_b, vals_b, out_b, zrow_b,
             sem):
        c = lax.axis_index("c")
        s = lax.axis_index("s")
        w = s * NC + c

        ones = jnp.ones((16,), jnp.float32)
        zv = jnp.zeros((16,), jnp.float32)

        @pl.loop(0, CHUNK // 16)
        def _fill1(i):
            ones_b[pl.ds(i * 16, 16)] = ones

        @pl.loop(0, ZCH // 16)
        def _fill0(i):
            zrow_b[pl.ds(i * 16, 16)] = zv

        @pl.loop(0, 2)
        def _zero(i):
            pltpu.sync_copy(zrow_b, cnt_sh.at[pl.ds(s * RPT + i * ZCH, ZCH)])

        plsc.subcore_barrier()

        def count_chunk(base, nb, ones_ref, idx_ref):
            pltpu.sync_copy(dst_h.at[pl.ds(base, nb)], dst_b.at[pl.ds(0, nb)])
            pltpu.sync_copy(typ_h.at[pl.ds(base, nb)], typ_b.at[pl.ds(0, nb)])
            for t in range(nb // 16):
                sl = pl.ds(t * 16, 16)
                idx_ref[sl] = typ_b[sl] * n_nodes + dst_b[sl]
            pltpu.sync_copy(ones_ref, cnt_sh.at[idx_ref], add=True)

        @pl.loop(0, n_full_cnt)
        def _count(i):
            count_chunk(s * EPT + i * CHUNK, CHUNK, ones_b, cidx_b)

        if tail_cnt:
            count_chunk(s * EPT + n_full_cnt * CHUNK, tail_cnt,
                        ones_b.at[pl.ds(0, tail_cnt)], cidx_tc)

        plsc.subcore_barrier()

        def emit_chunk(base, nb, idx_ref):
            pltpu.sync_copy(src_h.at[pl.ds(base, nb)], src_b.at[pl.ds(0, nb)])
            pltpu.sync_copy(dst_h.at[pl.ds(base, nb)], dst_b.at[pl.ds(0, nb)])
            pltpu.sync_copy(typ_h.at[pl.ds(base, nb)], typ_b.at[pl.ds(0, nb)])
            for t in range(nb // 16):
                sl = pl.ds(t * 16, 16)
                gidx_b[sl] = typ_b[sl] * n_nodes + src_b[sl]
                idx_ref[sl] = typ_b[sl] * n_nodes + dst_b[sl]
            pltpu.sync_copy(gidx_b.at[pl.ds(0, nb)], gidx_h.at[pl.ds(base, nb)])
            pltpu.async_copy(cnt_sh.at[idx_ref], vals_b.at[pl.ds(0, nb)],
                             sem).wait()
            for t in range(nb // 16):
                sl = pl.ds(t * 16, 16)
                out_b[sl] = 1.0 / jnp.maximum(vals_b[sl], 1.0)
            pltpu.sync_copy(out_b.at[pl.ds(0, nb)], invc_h.at[pl.ds(base, nb)])

        @pl.loop(0, n_full)
        def _emit(i):
            emit_chunk(w * EPW + i * CHUNK, CHUNK, cidx_b)

        if tail:
            emit_chunk(w * EPW + n_full * CHUNK, tail, cidx_te)

    kern = pl.kernel(
        body,
        out_type=(jax.ShapeDtypeStruct((E,), jnp.int32),
                  jax.ShapeDtypeStruct((E,), jnp.float32)),
        mesh=_sc_mesh(),
        scratch_types=(
            pltpu.VMEM_SHARED((PAD,), jnp.float32),
            pltpu.VMEM((CHUNK,), jnp.float32),
            pltpu.VMEM((CHUNK,), jnp.int32),
            pltpu.VMEM((CHUNK,), jnp.int32),
            pltpu.VMEM((CHUNK,), jnp.int32),
            pltpu.VMEM((CHUNK,), jnp.int32),
            pltpu.VMEM((max(tail_cnt, 8),), jnp.int32),
            pltpu.VMEM((max(tail, 8),), jnp.int32),
            pltpu.VMEM((CHUNK,), jnp.int32),
            pltpu.VMEM((CHUNK,), jnp.float32),
            pltpu.VMEM((CHUNK,), jnp.float32),
            pltpu.VMEM(((PAD // NS) // 2,), jnp.float32),
            pltpu.SemaphoreType.DMA,
        ),
    )
    return kern(src, dst, etype)


def _sc_aggregate(Y2, gidx, dst, invc, n_nodes):
    RN, C = Y2.shape
    E = gidx.shape[0]
    EPW = E // NW
    n_full, tail = EPW // CHUNK, EPW % CHUNK
    assert tail % 8 == 0
    NPT8 = 632
    LAST = n_nodes - (NS - 1) * NPT8
    T_MAIN = NPT8 - 4 * CHUNK
    T_LAST = LAST - 4 * CHUNK
    assert T_MAIN > 0 and T_LAST > 0 and T_MAIN % 8 == 0 and T_LAST % 8 == 0

    def body(y_h, gidx_h, dst_h, invc_h, out_h, agg_sh, rows0, rows1, gidx0,
             gidx1, didx0, didx1, sidx0, sidx1, invc0, invc1, rows_t, gidx_t,
             dst_t, invc_t, gsem0, gsem1, ssem0, ssem1, tsem):
        c = lax.axis_index("c")
        s = lax.axis_index("s")
        w = s * NC + c
        zv = jnp.zeros((16,), jnp.float32)
        zb = rows0
        rowss = (rows0, rows1)
        gidxs, didxs, sidxs = (gidx0, gidx1), (didx0, didx1), (sidx0, sidx1)
        invcs = (invc0, invc1)
        gsems, ssems = (gsem0, gsem1), (ssem0, ssem1)

        @pl.loop(0, CHUNK)
        def _fillz(i):
            for t in range(C // 16):
                zb[i, pl.ds(t * 16, 16)] = zv

        rbase = s * NPT8

        @pl.loop(0, 4)
        def _zero(i):
            pltpu.sync_copy(zb, agg_sh.at[pl.ds(rbase + i * CHUNK, CHUNK)])

        @pl.when(s < NS - 1)
        def _zt_main():
            pltpu.sync_copy(zb.at[pl.ds(0, T_MAIN)],
                            agg_sh.at[pl.ds(rbase + 4 * CHUNK, T_MAIN)])

        @pl.when(s == NS - 1)
        def _zt_last():
            pltpu.sync_copy(zb.at[pl.ds(0, T_LAST)],
                            agg_sh.at[pl.ds(rbase + 4 * CHUNK, T_LAST)])

        plsc.subcore_barrier()

        def meta(base, b):
            pltpu.sync_copy(gidx_h.at[pl.ds(base, CHUNK)], gidxs[b])
            pltpu.sync_copy(dst_h.at[pl.ds(base, CHUNK)], didxs[b])
            pltpu.sync_copy(invc_h.at[pl.ds(base, CHUNK)], invcs[b])

        def gdesc(b):
            return pltpu.make_async_copy(y_h.at[gidxs[b]], rowss[b], gsems[b])

        def sdesc(b):
            return pltpu.make_async_copy(rowss[b], agg_sh.at[sidxs[b]],
                                         ssems[b])

        def half(i, b):
            k = i + b
            gdesc(b).wait()

            @pl.loop(0, CHUNK // 16)
            def _scale(t):
                t16 = t * 16
                v = invcs[b][pl.ds(t16, 16)]
                sidxs[b][pl.ds(t16, 16)] = didxs[b][pl.ds(t16, 16)]
                for jj in range(16):
                    sv = v[jj]
                    for u in range(C // 16):
                        sl = pl.ds(u * 16, 16)
                        rowss[b][t16 + jj, sl] = rowss[b][t16 + jj, sl] * sv

            sdesc(b).start(add=True)

            @pl.when(k + 2 < n_full)
            def _pref():
                meta(w * EPW + (k + 2) * CHUNK, b)
                sdesc(b).wait()
                gdesc(b).start()

            @pl.when(k + 2 >= n_full)
            def _fin():
                sdesc(b).wait()

        meta(w * EPW, 0)
        gdesc(0).start()
        meta(w * EPW + CHUNK, 1)
        gdesc(1).start()

        @pl.loop(0, n_full, step=2)
        def _main(i):
            half(i, 0)
            half(i, 1)

        if tail:
            base = w * EPW + n_full * CHUNK
            pltpu.sync_copy(gidx_h.at[pl.ds(base, tail)], gidx_t)
            pltpu.sync_copy(dst_h.at[pl.ds(base, tail)], dst_t)
            pltpu.sync_copy(invc_h.at[pl.ds(base, tail)], invc_t)
            pltpu.async_copy(y_h.at[gidx_t], rows_t, tsem).wait()
            for t in range(tail // 16):
                v = invc_t[pl.ds(t * 16, 16)]
                for jj in range(16):
                    sv = v[jj]
                    j = t * 16 + jj
                    for u in range(C // 16):
                        sl = pl.ds(u * 16, 16)
                        rows_t[j, sl] = rows_t[j, sl] * sv
            pltpu.sync_copy(rows_t, agg_sh.at[dst_t], add=True)

        plsc.subcore_barrier()

        def dump(off, nrows):
            pltpu.sync_copy(agg_sh.at[pl.ds(rbase + off, nrows)],
                            zb.at[pl.ds(0, nrows)])
            pltpu.sync_copy(zb.at[pl.ds(0, nrows)],
                            out_h.at[pl.ds(c * n_nodes + rbase + off, nrows)])

        @pl.loop(0, 4)
        def _dump(i):
            dump(i * CHUNK, CHUNK)

        @pl.when(s < NS - 1)
        def _dt_main():
            dump(4 * CHUNK, T_MAIN)

        @pl.when(s == NS - 1)
        def _dt_last():
            dump(4 * CHUNK, T_LAST)

    kern = pl.kernel(
        body,
        out_type=jax.ShapeDtypeStruct((2 * n_nodes, C), jnp.float32),
        mesh=_sc_mesh(),
        scratch_types=(
            pltpu.VMEM_SHARED((n_nodes, C), jnp.float32),
            pltpu.VMEM((CHUNK, C), jnp.float32),
            pltpu.VMEM((CHUNK, C), jnp.float32),
            pltpu.VMEM((CHUNK,), jnp.int32),
            pltpu.VMEM((CHUNK,), jnp.int32),
            pltpu.VMEM((CHUNK,), jnp.int32),
            pltpu.VMEM((CHUNK,), jnp.int32),
            pltpu.VMEM((CHUNK,), jnp.int32),
            pltpu.VMEM((CHUNK,), jnp.int32),
            pltpu.VMEM((CHUNK,), jnp.float32),
            pltpu.VMEM((CHUNK,), jnp.float32),
            pltpu.VMEM((max(tail, 8), C), jnp.float32),
            pltpu.VMEM((max(tail, 8),), jnp.int32),
            pltpu.VMEM((max(tail, 8),), jnp.int32),
            pltpu.VMEM((max(tail, 8),), jnp.float32),
            pltpu.SemaphoreType.DMA,
            pltpu.SemaphoreType.DMA,
            pltpu.SemaphoreType.DMA,
            pltpu.SemaphoreType.DMA,
            pltpu.SemaphoreType.DMA,
        ),
    )
    return kern(Y2, gidx, dst, invc)


def _sc_gather_rows(lx, rx, vi1, vi2):
    N, C = lx.shape
    Vn = vi1.shape[0]
    VPW = Vn // NW

    def body(lx_h, rx_h, vi1_h, vi2_h, o1_h, o2_h, ib, rb, sem):
        c = lax.axis_index("c")
        s = lax.axis_index("s")
        w = s * NC + c
        base = w * VPW
        pltpu.sync_copy(vi1_h.at[pl.ds(base, VPW)], ib)
        pltpu.async_copy(lx_h.at[ib], rb, sem).wait()
        pltpu.sync_copy(rb, o1_h.at[pl.ds(base, VPW)])
        pltpu.sync_copy(vi2_h.at[pl.ds(base, VPW)], ib)
        pltpu.async_copy(rx_h.at[ib], rb, sem).wait()
        pltpu.sync_copy(rb, o2_h.at[pl.ds(base, VPW)])

    kern = pl.kernel(
        body,
        out_type=(jax.ShapeDtypeStruct((Vn, C), jnp.float32),
                  jax.ShapeDtypeStruct((Vn, C), jnp.float32)),
        mesh=_sc_mesh(),
        scratch_types=(
            pltpu.VMEM((VPW,), jnp.int32),
            pltpu.VMEM((VPW, C), jnp.float32),
            pltpu.SemaphoreType.DMA,
        ),
    )
    return kern(lx, rx, vi1, vi2)


def _run_side(x, edge_index, edge_type, W, root, bias, g, b, rounds):
    N, C = x.shape
    R = W.shape[0]
    src = edge_index[0]
    dst = edge_index[1]
    gidx, invc = _sc_prep(src, dst, edge_type, N, R)
    bias2d = bias.reshape(1, C)
    g2d = g.reshape(1, C)
    b2d = b.reshape(1, C)
    Y = _tc_transform(x, W)
    for t in range(rounds):
        parts = _sc_aggregate(Y.reshape(R * N, C), gidx, dst, invc, N)
        x, Y = _tc_combine(x, parts.reshape(2, N, C), root, bias2d, W, g2d,
                           b2d, want_y=(t < rounds - 1))
    return x


def kernel(left_x, right_x, left_edge_index, left_edge_type, right_edge_index,
           right_edge_type, varindex1, varindex2, W_left, root_left, bias_left,
           W_right, root_right, bias_right, ln_gamma, ln_beta):
    rounds = 5
    lx = _run_side(left_x, left_edge_index, left_edge_type, W_left, root_left,
                   bias_left, ln_gamma, ln_beta, rounds)
    rx = _run_side(right_x, right_edge_index, right_edge_type, W_right,
                   root_right, bias_right, ln_gamma, ln_beta, rounds)
    v1, v2 = _sc_gather_rows(lx, rx, varindex1, varindex2)
    dots = _tc_dot(v1, v2)
    return (dots, lx)
